# baseline (speedup 1.0000x reference)
import functools

import jax
import jax.numpy as jnp
from jax import lax
from jax.experimental import pallas as pl
from jax.experimental.pallas import tpu as pltpu
from jax.experimental.pallas import tpu_sc as plsc

N_I = 10000
N_D = 2000
E_II = 320000
E_ID = 10000
D = 128
L = 2

NC = 2
NS = 16
NW = NC * NS

C = 80
II_CHUNKS = E_II // (NW * C)
IB = 25
NBLK = II_CHUNKS // IB
CA = 128
EPT_PAD = 10240
IA = 10
NBLKA = EPT_PAD // CA // IA
N_PAD = 10240
ROWS_PER_TILE = N_PAD // NS
ZROWS = 8

DEG_PAD = N_PAD
DEG_W = 16
DEG_ROWS_PER_TILE = DEG_PAD // NS

ID_PAD = 10240
ID_CHUNKS = ID_PAD // (NW * C)
ND_PAD = 2048
ND_ROWS_PER_TILE = ND_PAD // NS

_MESH = plsc.VectorSubcoreMesh(core_axis_name="c", subcore_axis_name="s")


def _zero_rows(ref, nrows, width):
    @pl.loop(0, nrows)
    def _(i):
        for k in range(width // 16):
            ref[i, pl.ds(k * 16, 16)] = jnp.zeros((16,), jnp.float32)


@functools.partial(
    pl.kernel,
    out_type=jax.ShapeDtypeStruct((NC, N_PAD, D), jnp.float32),
    mesh=_MESH,
    scratch_types=[
        pltpu.VMEM((IB, C), jnp.int32),
        pltpu.VMEM((C, D), jnp.float32),
        pltpu.VMEM((ZROWS, D), jnp.float32),
        pltpu.VMEM_SHARED((N_PAD, D), jnp.float32),
    ],
)
def _deg_kernel(dst_hbm, out_hbm, dst_v, ones_v, zbuf_v, acc_sh):
    c = lax.axis_index("c")
    s = lax.axis_index("s")
    wid = c * NS + s

    @pl.loop(0, C)
    def _(i):
        for k in range(D // 16):
            ones_v[i, pl.ds(k * 16, 16)] = jnp.ones((16,), jnp.float32)

    _zero_rows(zbuf_v, ZROWS, D)
    base = s * ROWS_PER_TILE

    @pl.loop(0, ROWS_PER_TILE // ZROWS)
    def _(t):
        pltpu.sync_copy(zbuf_v, acc_sh.at[pl.ds(base + t * ZROWS, ZROWS)])

    plsc.subcore_barrier()

    for blk in range(NBLK):
        pltpu.sync_copy(dst_hbm.at[wid, blk], dst_v)

        @pl.loop(0, IB)
        def _(j):
            pltpu.sync_copy(ones_v, acc_sh.at[dst_v.at[j]], add=True)

    plsc.subcore_barrier()
    pltpu.sync_copy(acc_sh.at[pl.ds(base, ROWS_PER_TILE)],
                    out_hbm.at[c, pl.ds(base, ROWS_PER_TILE)])


@functools.partial(
    pl.kernel,
    out_type=jax.ShapeDtypeStruct((NC, N_PAD, D), jnp.float32),
    mesh=_MESH,
    scratch_types=[
        pltpu.VMEM((IA, CA), jnp.int32),
        pltpu.VMEM((IA, CA), jnp.int32),
        pltpu.VMEM((2, CA, D), jnp.float32),
        pltpu.VMEM((ZROWS, D), jnp.float32),
        pltpu.VMEM_SHARED((N_PAD, D), jnp.float32),
        pltpu.SemaphoreType.DMA,
        pltpu.SemaphoreType.DMA,
    ],
)
def _agg_kernel(h_hbm, src_hbm, dst_hbm, out_hbm,
                src_v, dst_v, rows_v, zbuf_v, acc_sh, sem0, sem1):
    c = lax.axis_index("c")
    s = lax.axis_index("s")
    wid = c * NS + s

    _zero_rows(zbuf_v, ZROWS, D)
    base = s * ROWS_PER_TILE

    @pl.loop(0, ROWS_PER_TILE // ZROWS)
    def _(t):
        pltpu.sync_copy(zbuf_v, acc_sh.at[pl.ds(base + t * ZROWS, ZROWS)])

    plsc.subcore_barrier()

    sems = (sem0, sem1)
    for blk in range(NBLKA):
        pltpu.sync_copy(src_hbm.at[wid, blk], src_v)
        pltpu.sync_copy(dst_hbm.at[wid, blk], dst_v)
        pltpu.async_copy(h_hbm.at[src_v.at[0]], rows_v.at[0], sem0)
        pltpu.async_copy(h_hbm.at[src_v.at[1]], rows_v.at[1], sem1)

        @pl.loop(0, IA, step=2)
        def _(j):
            for b in range(2):
                jj = j + b
                pltpu.make_async_copy(h_hbm.at[src_v.at[jj]],
                                      rows_v.at[b], sems[b]).wait()
                pltpu.sync_copy(rows_v.at[b], acc_sh.at[dst_v.at[jj]], add=True)

                @pl.when(jj + 2 < IA)
                def _():
                    pltpu.async_copy(h_hbm.at[src_v.at[jj + 2]],
                                     rows_v.at[b], sems[b])

    plsc.subcore_barrier()
    pltpu.sync_copy(acc_sh.at[pl.ds(base, ROWS_PER_TILE)],
                    out_hbm.at[c, pl.ds(base, ROWS_PER_TILE)])


@functools.partial(
    pl.kernel,
    out_type=jax.ShapeDtypeStruct((NC, ND_PAD, D), jnp.float32),
    mesh=_MESH,
    scratch_types=[
        pltpu.VMEM((ID_CHUNKS, C), jnp.int32),
        pltpu.VMEM((ID_CHUNKS, C), jnp.int32),
        pltpu.VMEM((C, D), jnp.float32),
        pltpu.VMEM((ND_ROWS_PER_TILE, D), jnp.float32),
        pltpu.VMEM_SHARED((ND_PAD, D), jnp.float32),
        pltpu.SemaphoreType.DMA,
    ],
)
def _i2d_kernel(h_hbm, src_hbm, dst_hbm, out_hbm,
                src_v, dst_v, rows_v, zbuf_v, acc_sh, sem):
    c = lax.axis_index("c")
    s = lax.axis_index("s")
    wid = c * NS + s

    _zero_rows(zbuf_v, ND_ROWS_PER_TILE, D)
    base = s * ND_ROWS_PER_TILE
    pltpu.sync_copy(zbuf_v, acc_sh.at[pl.ds(base, ND_ROWS_PER_TILE)])
    pltpu.sync_copy(src_hbm.at[wid], src_v)
    pltpu.sync_copy(dst_hbm.at[wid], dst_v)
    plsc.subcore_barrier()

    @pl.loop(0, ID_CHUNKS)
    def _(j):
        pltpu.async_copy(h_hbm.at[src_v.at[j]], rows_v, sem).wait()
        pltpu.sync_copy(rows_v, acc_sh.at[dst_v.at[j]], add=True)

    plsc.subcore_barrier()
    pltpu.sync_copy(acc_sh.at[pl.ds(base, ND_ROWS_PER_TILE)],
                    out_hbm.at[c, pl.ds(base, ND_ROWS_PER_TILE)])


RB = 2048


def _prep_body(deg_ref, r_ref, norm_ref, rn_ref):
    deg = deg_ref[0, :, 0:1] + deg_ref[1, :, 0:1]
    nrm = lax.rsqrt(jnp.maximum(deg, 1.0))
    norm_ref[...] = nrm
    rn_ref[...] = r_ref[...] * nrm


def _prep_call(deg_p, r_node):
    return pl.pallas_call(
        _prep_body,
        out_shape=(jax.ShapeDtypeStruct((N_PAD, 1), jnp.float32),
                   jax.ShapeDtypeStruct((N_PAD, D), jnp.float32)),
    )(deg_p, r_node)


def _gcn_r_body(p_ref, norm_ref, w_ref, b_ref, in_ref, rn_out, isc_out):
    nrm = norm_ref[...]
    agg = (p_ref[0] + p_ref[1]) * nrm
    r = jnp.dot(agg, w_ref[...], preferred_element_type=jnp.float32) + b_ref[...]
    r = jnp.maximum(r, 0.0)
    rn_out[...] = r * nrm
    isc_out[...] = (in_ref[...] + r) * nrm


def _gcn_r_call(p, norm, w, b, i_node):
    return pl.pallas_call(
        _gcn_r_body,
        grid=(N_PAD // RB,),
        in_specs=[
            pl.BlockSpec((NC, RB, D), lambda i: (0, i, 0)),
            pl.BlockSpec((RB, 1), lambda i: (i, 0)),
            pl.BlockSpec((D, D), lambda i: (0, 0)),
            pl.BlockSpec((1, D), lambda i: (0, 0)),
            pl.BlockSpec((RB, D), lambda i: (i, 0)),
        ],
        out_specs=(pl.BlockSpec((RB, D), lambda i: (i, 0)),
                   pl.BlockSpec((RB, D), lambda i: (i, 0))),
        out_shape=(jax.ShapeDtypeStruct((N_PAD, D), jnp.float32),
                   jax.ShapeDtypeStruct((N_PAD, D), jnp.float32)),
    )(p, norm, w, b, i_node)


def _gcn_i_body(p_ref, norm_ref, w_ref, b_ref, i_out):
    agg = (p_ref[0] + p_ref[1]) * norm_ref[...]
    r = jnp.dot(agg, w_ref[...], preferred_element_type=jnp.float32) + b_ref[...]
    i_out[...] = jnp.maximum(r, 0.0)


def _gcn_i_call(p, norm, w, b):
    return pl.pallas_call(
        _gcn_i_body,
        grid=(N_PAD // RB,),
        in_specs=[
            pl.BlockSpec((NC, RB, D), lambda i: (0, i, 0)),
            pl.BlockSpec((RB, 1), lambda i: (i, 0)),
            pl.BlockSpec((D, D), lambda i: (0, 0)),
            pl.BlockSpec((1, D), lambda i: (0, 0)),
        ],
        out_specs=pl.BlockSpec((RB, D), lambda i: (i, 0)),
        out_shape=jax.ShapeDtypeStruct((N_PAD, D), jnp.float32),
    )(p, norm, w, b)


def _leaky(x):
    return jnp.where(x >= 0, x, 0.01 * x)


def _mlp_body(p_ref, w1, b1, w2, b2, w3, b3, out_ref):
    d = p_ref[0, :N_D, :] + p_ref[1, :N_D, :]
    h = _leaky(jnp.dot(d, w1[...], preferred_element_type=jnp.float32) + b1[...])
    h = _leaky(jnp.dot(h, w2[...], preferred_element_type=jnp.float32) + b2[...])
    out_ref[...] = jnp.dot(h, w3[...], preferred_element_type=jnp.float32) + b3[...]


def _mlp_call(d_p, w1, b1, w2, b2, w3, b3):
    return pl.pallas_call(
        _mlp_body,
        out_shape=jax.ShapeDtypeStruct((N_D, 1), jnp.float32),
    )(d_p, w1, b1, w2, b2, w3, b3)


def kernel(r_node, i_node, r2r_edge, d2d_edge, i2i_src, i2i_dst, i2d_src, i2d_dst,
           W_r2r, b_r2r, W_i2i, b_i2i, W_s1, b_s1, W_s2, b_s2, W_s3, b_s3):
    r_node = jnp.pad(r_node, ((0, N_PAD - N_I), (0, 0)))
    i_node = jnp.pad(i_node, ((0, N_PAD - N_I), (0, 0)))
    dst_r = i2i_dst.reshape(NW, NBLK, IB, C)
    pad_e = EPT_PAD - E_II // NW
    src_a = jnp.pad(i2i_src.reshape(NW, E_II // NW), ((0, 0), (0, pad_e))
                    ).reshape(NW, NBLKA, IA, CA)
    dst_a = jnp.pad(i2i_dst.reshape(NW, E_II // NW), ((0, 0), (0, pad_e)),
                    constant_values=N_PAD - 1).reshape(NW, NBLKA, IA, CA)
    pad = ID_PAD - E_ID
    id_src = jnp.concatenate([i2d_src, jnp.zeros((pad,), jnp.int32)]
                             ).reshape(NW, ID_CHUNKS, C)
    id_dst = jnp.concatenate([i2d_dst, jnp.full((pad,), N_D, jnp.int32)]
                             ).reshape(NW, ID_CHUNKS, C)

    deg_p = _deg_kernel(dst_r)
    norm, h = _prep_call(deg_p, r_node)

    for l in range(L):
        p = _agg_kernel(h, src_a, dst_a)
        h, hi = _gcn_r_call(p, norm, W_r2r[l], b_r2r[l].reshape(1, D), i_node)
        p = _agg_kernel(hi, src_a, dst_a)
        i_node = _gcn_i_call(p, norm, W_i2i[l], b_i2i[l].reshape(1, D))

    d_p = _i2d_kernel(i_node, id_src, id_dst)
    return _mlp_call(d_p, W_s1, b_s1.reshape(1, -1), W_s2, b_s2.reshape(1, -1),
                     W_s3, b_s3.reshape(1, -1))


# speedup vs baseline: 2.9787x; 2.9787x over previous
import functools

import jax
import jax.numpy as jnp
from jax import lax
from jax.experimental import pallas as pl
from jax.experimental.pallas import tpu as pltpu
from jax.experimental.pallas import tpu_sc as plsc

N_I = 10000
N_D = 2000
E_II = 320000
E_ID = 10000
D = 128
L = 2

NC = 2
NS = 16
NW = NC * NS

C = 80
II_CHUNKS = E_II // (NW * C)
IB = 25
NBLK = II_CHUNKS // IB
N_PAD = 10240
ROWS_PER_TILE = N_PAD // NS
ZROWS = 8

DEG_PAD = N_PAD
DEG_W = 16
DEG_ROWS_PER_TILE = DEG_PAD // NS

ID_PAD = 10240
ID_CHUNKS = ID_PAD // (NW * C)
ND_PAD = 2048
ND_ROWS_PER_TILE = ND_PAD // NS

_MESH = plsc.VectorSubcoreMesh(core_axis_name="c", subcore_axis_name="s")


def _zero_rows(ref, nrows, width):
    @pl.loop(0, nrows)
    def _(i):
        for k in range(width // 16):
            ref[i, pl.ds(k * 16, 16)] = jnp.zeros((16,), jnp.float32)


@functools.partial(
    pl.kernel,
    out_type=jax.ShapeDtypeStruct((NC, N_PAD, D), jnp.float32),
    mesh=_MESH,
    scratch_types=[
        pltpu.VMEM((IB, C), jnp.int32),
        pltpu.VMEM((C, D), jnp.float32),
        pltpu.VMEM((ZROWS, D), jnp.float32),
        pltpu.VMEM_SHARED((N_PAD, D), jnp.float32),
        pltpu.SemaphoreType.DMA,
        pltpu.SemaphoreType.DMA,
    ],
)
def _deg_kernel(dst_hbm, out_hbm, dst_v, ones_v, zbuf_v, acc_sh, dsem0, dsem1):
    c = lax.axis_index("c")
    s = lax.axis_index("s")
    wid = c * NS + s

    @pl.loop(0, C)
    def _(i):
        for k in range(D // 16):
            ones_v[i, pl.ds(k * 16, 16)] = jnp.ones((16,), jnp.float32)

    _zero_rows(zbuf_v, ZROWS, D)
    base = s * ROWS_PER_TILE

    @pl.loop(0, ROWS_PER_TILE // ZROWS)
    def _(t):
        pltpu.sync_copy(zbuf_v, acc_sh.at[pl.ds(base + t * ZROWS, ZROWS)])

    plsc.subcore_barrier()

    dsems = (dsem0, dsem1)
    for blk in range(NBLK):
        pltpu.sync_copy(dst_hbm.at[wid, blk], dst_v)
        pltpu.async_copy(ones_v, acc_sh.at[dst_v.at[0]], dsem0, add=True)
        pltpu.async_copy(ones_v, acc_sh.at[dst_v.at[1]], dsem1, add=True)

        @pl.loop(0, IB - 1, step=2)
        def _(j):
            for b in range(2):
                jj = j + b
                pltpu.make_async_copy(ones_v, acc_sh.at[dst_v.at[jj]],
                                      dsems[b]).wait()

                @pl.when(jj + 2 < IB)
                def _():
                    pltpu.async_copy(ones_v, acc_sh.at[dst_v.at[jj + 2]],
                                     dsems[b], add=True)

        jt = IB - 1
        pltpu.make_async_copy(ones_v, acc_sh.at[dst_v.at[jt]], dsem0).wait()

    plsc.subcore_barrier()
    pltpu.sync_copy(acc_sh.at[pl.ds(base, ROWS_PER_TILE)],
                    out_hbm.at[c, pl.ds(base, ROWS_PER_TILE)])


@functools.partial(
    pl.kernel,
    out_type=jax.ShapeDtypeStruct((NC, N_PAD, D), jnp.float32),
    mesh=_MESH,
    scratch_types=[
        pltpu.VMEM((IB, C), jnp.int32),
        pltpu.VMEM((IB, C), jnp.int32),
        pltpu.VMEM((3, C, D), jnp.float32),
        pltpu.VMEM((ZROWS, D), jnp.float32),
        pltpu.VMEM_SHARED((N_PAD, D), jnp.float32),
        pltpu.SemaphoreType.DMA,
        pltpu.SemaphoreType.DMA,
        pltpu.SemaphoreType.DMA,
    ],
)
def _agg_kernel(h_hbm, src_hbm, dst_hbm, out_hbm,
                src_v, dst_v, rows_v, zbuf_v, acc_sh, sem0, sem1, sem2):
    c = lax.axis_index("c")
    s = lax.axis_index("s")
    wid = c * NS + s

    _zero_rows(zbuf_v, ZROWS, D)
    base = s * ROWS_PER_TILE

    @pl.loop(0, ROWS_PER_TILE // ZROWS)
    def _(t):
        pltpu.sync_copy(zbuf_v, acc_sh.at[pl.ds(base + t * ZROWS, ZROWS)])

    plsc.subcore_barrier()

    NB = 3
    sems = (sem0, sem1, sem2)
    for blk in range(NBLK):
        pltpu.sync_copy(src_hbm.at[wid, blk], src_v)
        pltpu.sync_copy(dst_hbm.at[wid, blk], dst_v)
        for b in range(NB):
            pltpu.async_copy(h_hbm.at[src_v.at[b]], rows_v.at[b], sems[b])

        @pl.loop(0, IB - 1, step=NB)
        def _(j):
            for b in range(NB):
                jj = j + b
                pltpu.make_async_copy(h_hbm.at[src_v.at[jj]],
                                      rows_v.at[b], sems[b]).wait()
                pltpu.sync_copy(rows_v.at[b], acc_sh.at[dst_v.at[jj]], add=True)

                @pl.when(jj + NB < IB)
                def _():
                    pltpu.async_copy(h_hbm.at[src_v.at[jj + NB]],
                                     rows_v.at[b], sems[b])

        jt = IB - 1
        pltpu.make_async_copy(h_hbm.at[src_v.at[jt]], rows_v.at[0], sem0).wait()
        pltpu.sync_copy(rows_v.at[0], acc_sh.at[dst_v.at[jt]], add=True)

    plsc.subcore_barrier()
    pltpu.sync_copy(acc_sh.at[pl.ds(base, ROWS_PER_TILE)],
                    out_hbm.at[c, pl.ds(base, ROWS_PER_TILE)])


@functools.partial(
    pl.kernel,
    out_type=jax.ShapeDtypeStruct((NC, ND_PAD, D), jnp.float32),
    mesh=_MESH,
    scratch_types=[
        pltpu.VMEM((ID_CHUNKS, C), jnp.int32),
        pltpu.VMEM((ID_CHUNKS, C), jnp.int32),
        pltpu.VMEM((C, D), jnp.float32),
        pltpu.VMEM((ND_ROWS_PER_TILE, D), jnp.float32),
        pltpu.VMEM_SHARED((ND_PAD, D), jnp.float32),
        pltpu.SemaphoreType.DMA,
    ],
)
def _i2d_kernel(h_hbm, src_hbm, dst_hbm, out_hbm,
                src_v, dst_v, rows_v, zbuf_v, acc_sh, sem):
    c = lax.axis_index("c")
    s = lax.axis_index("s")
    wid = c * NS + s

    _zero_rows(zbuf_v, ND_ROWS_PER_TILE, D)
    base = s * ND_ROWS_PER_TILE
    pltpu.sync_copy(zbuf_v, acc_sh.at[pl.ds(base, ND_ROWS_PER_TILE)])
    pltpu.sync_copy(src_hbm.at[wid], src_v)
    pltpu.sync_copy(dst_hbm.at[wid], dst_v)
    plsc.subcore_barrier()

    @pl.loop(0, ID_CHUNKS)
    def _(j):
        pltpu.async_copy(h_hbm.at[src_v.at[j]], rows_v, sem).wait()
        pltpu.sync_copy(rows_v, acc_sh.at[dst_v.at[j]], add=True)

    plsc.subcore_barrier()
    pltpu.sync_copy(acc_sh.at[pl.ds(base, ND_ROWS_PER_TILE)],
                    out_hbm.at[c, pl.ds(base, ND_ROWS_PER_TILE)])


RB = 2048


def _prep_body(deg_ref, r_ref, norm_ref, rn_ref):
    deg = deg_ref[0, :, 0:1] + deg_ref[1, :, 0:1]
    nrm = lax.rsqrt(jnp.maximum(deg, 1.0))
    norm_ref[...] = nrm
    rn_ref[...] = r_ref[...] * nrm


def _prep_call(deg_p, r_node):
    return pl.pallas_call(
        _prep_body,
        out_shape=(jax.ShapeDtypeStruct((N_PAD, 1), jnp.float32),
                   jax.ShapeDtypeStruct((N_PAD, D), jnp.float32)),
    )(deg_p, r_node)


def _gcn_r_body(p_ref, norm_ref, w_ref, b_ref, in_ref, rn_out, isc_out):
    nrm = norm_ref[...]
    agg = (p_ref[0] + p_ref[1]) * nrm
    r = jnp.dot(agg, w_ref[...], preferred_element_type=jnp.float32) + b_ref[...]
    r = jnp.maximum(r, 0.0)
    rn_out[...] = r * nrm
    isc_out[...] = (in_ref[...] + r) * nrm


def _gcn_r_call(p, norm, w, b, i_node):
    return pl.pallas_call(
        _gcn_r_body,
        grid=(N_PAD // RB,),
        in_specs=[
            pl.BlockSpec((NC, RB, D), lambda i: (0, i, 0)),
            pl.BlockSpec((RB, 1), lambda i: (i, 0)),
            pl.BlockSpec((D, D), lambda i: (0, 0)),
            pl.BlockSpec((1, D), lambda i: (0, 0)),
            pl.BlockSpec((RB, D), lambda i: (i, 0)),
        ],
        out_specs=(pl.BlockSpec((RB, D), lambda i: (i, 0)),
                   pl.BlockSpec((RB, D), lambda i: (i, 0))),
        out_shape=(jax.ShapeDtypeStruct((N_PAD, D), jnp.float32),
                   jax.ShapeDtypeStruct((N_PAD, D), jnp.float32)),
    )(p, norm, w, b, i_node)


def _gcn_i_body(p_ref, norm_ref, w_ref, b_ref, i_out):
    agg = (p_ref[0] + p_ref[1]) * norm_ref[...]
    r = jnp.dot(agg, w_ref[...], preferred_element_type=jnp.float32) + b_ref[...]
    i_out[...] = jnp.maximum(r, 0.0)


def _gcn_i_call(p, norm, w, b):
    return pl.pallas_call(
        _gcn_i_body,
        grid=(N_PAD // RB,),
        in_specs=[
            pl.BlockSpec((NC, RB, D), lambda i: (0, i, 0)),
            pl.BlockSpec((RB, 1), lambda i: (i, 0)),
            pl.BlockSpec((D, D), lambda i: (0, 0)),
            pl.BlockSpec((1, D), lambda i: (0, 0)),
        ],
        out_specs=pl.BlockSpec((RB, D), lambda i: (i, 0)),
        out_shape=jax.ShapeDtypeStruct((N_PAD, D), jnp.float32),
    )(p, norm, w, b)


def _leaky(x):
    return jnp.where(x >= 0, x, 0.01 * x)


def _mlp_body(p_ref, w1, b1, w2, b2, w3, b3, out_ref):
    d = p_ref[0, :N_D, :] + p_ref[1, :N_D, :]
    h = _leaky(jnp.dot(d, w1[...], preferred_element_type=jnp.float32) + b1[...])
    h = _leaky(jnp.dot(h, w2[...], preferred_element_type=jnp.float32) + b2[...])
    out_ref[...] = jnp.dot(h, w3[...], preferred_element_type=jnp.float32) + b3[...]


def _mlp_call(d_p, w1, b1, w2, b2, w3, b3):
    return pl.pallas_call(
        _mlp_body,
        out_shape=jax.ShapeDtypeStruct((N_D, 1), jnp.float32),
    )(d_p, w1, b1, w2, b2, w3, b3)


def kernel(r_node, i_node, r2r_edge, d2d_edge, i2i_src, i2i_dst, i2d_src, i2d_dst,
           W_r2r, b_r2r, W_i2i, b_i2i, W_s1, b_s1, W_s2, b_s2, W_s3, b_s3):
    r_node = jnp.pad(r_node, ((0, N_PAD - N_I), (0, 0)))
    i_node = jnp.pad(i_node, ((0, N_PAD - N_I), (0, 0)))
    src_r = i2i_src.reshape(NW, NBLK, IB, C)
    dst_r = i2i_dst.reshape(NW, NBLK, IB, C)
    pad = ID_PAD - E_ID
    id_src = jnp.concatenate([i2d_src, jnp.zeros((pad,), jnp.int32)]
                             ).reshape(NW, ID_CHUNKS, C)
    id_dst = jnp.concatenate([i2d_dst, jnp.full((pad,), N_D, jnp.int32)]
                             ).reshape(NW, ID_CHUNKS, C)

    deg_p = _deg_kernel(dst_r)
    norm, h = _prep_call(deg_p, r_node)

    for l in range(L):
        p = _agg_kernel(h, src_r, dst_r)
        h, hi = _gcn_r_call(p, norm, W_r2r[l], b_r2r[l].reshape(1, D), i_node)
        p = _agg_kernel(hi, src_r, dst_r)
        i_node = _gcn_i_call(p, norm, W_i2i[l], b_i2i[l].reshape(1, D))

    d_p = _i2d_kernel(i_node, id_src, id_dst)
    return _mlp_call(d_p, W_s1, b_s1.reshape(1, -1), W_s2, b_s2.reshape(1, -1),
                     W_s3, b_s3.reshape(1, -1))


# speedup vs baseline: 2.9809x; 1.0007x over previous
import functools

import jax
import jax.numpy as jnp
from jax import lax
from jax.experimental import pallas as pl
from jax.experimental.pallas import tpu as pltpu
from jax.experimental.pallas import tpu_sc as plsc

N_I = 10000
N_D = 2000
E_II = 320000
E_ID = 10000
D = 128
L = 2

NC = 2
NS = 16
NW = NC * NS

C = 80
II_CHUNKS = E_II // (NW * C)
IB = 25
NBLK = II_CHUNKS // IB
N_PAD = 10240
ROWS_PER_TILE = N_PAD // NS
ZROWS = 8

ID_PAD = 10240
ID_CHUNKS = ID_PAD // (NW * C)
ND_PAD = 2048
ND_ROWS_PER_TILE = ND_PAD // NS

_MESH = plsc.VectorSubcoreMesh(core_axis_name="c", subcore_axis_name="s")


def _zero_rows(ref, nrows, width):
    @pl.loop(0, nrows)
    def _(i):
        for k in range(width // 16):
            ref[i, pl.ds(k * 16, 16)] = jnp.zeros((16,), jnp.float32)


@functools.partial(
    pl.kernel,
    out_type=jax.ShapeDtypeStruct((NC, N_PAD, D), jnp.float32),
    mesh=_MESH,
    scratch_types=[
        pltpu.VMEM((IB, C), jnp.int32),
        pltpu.VMEM((C, D), jnp.float32),
        pltpu.VMEM((ZROWS, D), jnp.float32),
        pltpu.VMEM_SHARED((N_PAD, D), jnp.float32),
        pltpu.SemaphoreType.DMA,
        pltpu.SemaphoreType.DMA,
    ],
)
def _deg_kernel(dst_hbm, out_hbm, dst_v, ones_v, zbuf_v, acc_sh, dsem0, dsem1):
    c = lax.axis_index("c")
    s = lax.axis_index("s")
    wid = c * NS + s

    @pl.loop(0, C)
    def _(i):
        for k in range(D // 16):
            ones_v[i, pl.ds(k * 16, 16)] = jnp.ones((16,), jnp.float32)

    _zero_rows(zbuf_v, ZROWS, D)
    base = s * ROWS_PER_TILE

    @pl.loop(0, ROWS_PER_TILE // ZROWS)
    def _(t):
        pltpu.sync_copy(zbuf_v, acc_sh.at[pl.ds(base + t * ZROWS, ZROWS)])

    plsc.subcore_barrier()

    dsems = (dsem0, dsem1)
    for blk in range(NBLK):
        pltpu.sync_copy(dst_hbm.at[wid, blk], dst_v)
        pltpu.async_copy(ones_v, acc_sh.at[dst_v.at[0]], dsem0, add=True)
        pltpu.async_copy(ones_v, acc_sh.at[dst_v.at[1]], dsem1, add=True)

        @pl.loop(0, IB - 1, step=2)
        def _(j):
            for b in range(2):
                jj = j + b
                pltpu.make_async_copy(ones_v, acc_sh.at[dst_v.at[jj]],
                                      dsems[b]).wait()

                @pl.when(jj + 2 < IB)
                def _():
                    pltpu.async_copy(ones_v, acc_sh.at[dst_v.at[jj + 2]],
                                     dsems[b], add=True)

        jt = IB - 1
        pltpu.make_async_copy(ones_v, acc_sh.at[dst_v.at[jt]], dsem0).wait()

    plsc.subcore_barrier()
    pltpu.sync_copy(acc_sh.at[pl.ds(base, ROWS_PER_TILE)],
                    out_hbm.at[c, pl.ds(base, ROWS_PER_TILE)])


@functools.partial(
    pl.kernel,
    out_type=jax.ShapeDtypeStruct((NC, N_PAD, D), jnp.float32),
    mesh=_MESH,
    scratch_types=[
        pltpu.VMEM((IB, C), jnp.int32),
        pltpu.VMEM((IB, C), jnp.int32),
        pltpu.VMEM((3, C, D), jnp.float32),
        pltpu.VMEM((ZROWS, D), jnp.float32),
        pltpu.VMEM_SHARED((N_PAD, D), jnp.float32),
        pltpu.SemaphoreType.DMA,
        pltpu.SemaphoreType.DMA,
        pltpu.SemaphoreType.DMA,
    ],
)
def _agg_kernel(h_hbm, src_hbm, dst_hbm, out_hbm,
                src_v, dst_v, rows_v, zbuf_v, acc_sh, sem0, sem1, sem2):
    c = lax.axis_index("c")
    s = lax.axis_index("s")
    wid = c * NS + s

    _zero_rows(zbuf_v, ZROWS, D)
    base = s * ROWS_PER_TILE

    @pl.loop(0, ROWS_PER_TILE // ZROWS)
    def _(t):
        pltpu.sync_copy(zbuf_v, acc_sh.at[pl.ds(base + t * ZROWS, ZROWS)])

    plsc.subcore_barrier()

    NB = 3
    sems = (sem0, sem1, sem2)
    for blk in range(NBLK):
        pltpu.sync_copy(src_hbm.at[wid, blk], src_v)
        pltpu.sync_copy(dst_hbm.at[wid, blk], dst_v)
        for b in range(NB):
            pltpu.async_copy(h_hbm.at[src_v.at[b]], rows_v.at[b], sems[b])

        @pl.loop(0, IB - 1, step=NB)
        def _(j):
            for b in range(NB):
                jj = j + b
                pltpu.make_async_copy(h_hbm.at[src_v.at[jj]],
                                      rows_v.at[b], sems[b]).wait()
                pltpu.sync_copy(rows_v.at[b], acc_sh.at[dst_v.at[jj]], add=True)

                @pl.when(jj + NB < IB)
                def _():
                    pltpu.async_copy(h_hbm.at[src_v.at[jj + NB]],
                                     rows_v.at[b], sems[b])

        jt = IB - 1
        pltpu.make_async_copy(h_hbm.at[src_v.at[jt]], rows_v.at[0], sem0).wait()
        pltpu.sync_copy(rows_v.at[0], acc_sh.at[dst_v.at[jt]], add=True)

    plsc.subcore_barrier()
    pltpu.sync_copy(acc_sh.at[pl.ds(base, ROWS_PER_TILE)],
                    out_hbm.at[c, pl.ds(base, ROWS_PER_TILE)])


@functools.partial(
    pl.kernel,
    out_type=jax.ShapeDtypeStruct((NC, ND_PAD, D), jnp.float32),
    mesh=_MESH,
    scratch_types=[
        pltpu.VMEM((ID_CHUNKS, C), jnp.int32),
        pltpu.VMEM((ID_CHUNKS, C), jnp.int32),
        pltpu.VMEM((C, D), jnp.float32),
        pltpu.VMEM((ND_ROWS_PER_TILE, D), jnp.float32),
        pltpu.VMEM_SHARED((ND_PAD, D), jnp.float32),
        pltpu.SemaphoreType.DMA,
    ],
)
def _i2d_kernel(h_hbm, src_hbm, dst_hbm, out_hbm,
                src_v, dst_v, rows_v, zbuf_v, acc_sh, sem):
    c = lax.axis_index("c")
    s = lax.axis_index("s")
    wid = c * NS + s

    _zero_rows(zbuf_v, ND_ROWS_PER_TILE, D)
    base = s * ND_ROWS_PER_TILE
    pltpu.sync_copy(zbuf_v, acc_sh.at[pl.ds(base, ND_ROWS_PER_TILE)])
    pltpu.sync_copy(src_hbm.at[wid], src_v)
    pltpu.sync_copy(dst_hbm.at[wid], dst_v)
    plsc.subcore_barrier()

    @pl.loop(0, ID_CHUNKS)
    def _(j):
        pltpu.async_copy(h_hbm.at[src_v.at[j]], rows_v, sem).wait()
        pltpu.sync_copy(rows_v, acc_sh.at[dst_v.at[j]], add=True)

    plsc.subcore_barrier()
    pltpu.sync_copy(acc_sh.at[pl.ds(base, ND_ROWS_PER_TILE)],
                    out_hbm.at[c, pl.ds(base, ND_ROWS_PER_TILE)])


RB = 2048


def _prep_body(deg_ref, r_ref, norm_ref, rn_ref):
    deg = deg_ref[0, :, 0:1] + deg_ref[1, :, 0:1]
    nrm = lax.rsqrt(jnp.maximum(deg, 1.0))
    norm_ref[...] = nrm
    rn_ref[...] = r_ref[...] * nrm


def _prep_call(deg_p, r_node):
    return pl.pallas_call(
        _prep_body,
        out_shape=(jax.ShapeDtypeStruct((N_PAD, 1), jnp.float32),
                   jax.ShapeDtypeStruct((N_PAD, D), jnp.float32)),
    )(deg_p, r_node)


def _gcn_r_body(p_ref, norm_ref, w_ref, b_ref, in_ref, rn_out, isc_out):
    nrm = norm_ref[...]
    agg = (p_ref[0] + p_ref[1]) * nrm
    r = jnp.dot(agg, w_ref[...], preferred_element_type=jnp.float32) + b_ref[...]
    r = jnp.maximum(r, 0.0)
    rn_out[...] = r * nrm
    isc_out[...] = (in_ref[...] + r) * nrm


def _gcn_r_call(p, norm, w, b, i_node):
    return pl.pallas_call(
        _gcn_r_body,
        grid=(N_PAD // RB,),
        in_specs=[
            pl.BlockSpec((NC, RB, D), lambda i: (0, i, 0)),
            pl.BlockSpec((RB, 1), lambda i: (i, 0)),
            pl.BlockSpec((D, D), lambda i: (0, 0)),
            pl.BlockSpec((1, D), lambda i: (0, 0)),
            pl.BlockSpec((RB, D), lambda i: (i, 0)),
        ],
        out_specs=(pl.BlockSpec((RB, D), lambda i: (i, 0)),
                   pl.BlockSpec((RB, D), lambda i: (i, 0))),
        out_shape=(jax.ShapeDtypeStruct((N_PAD, D), jnp.float32),
                   jax.ShapeDtypeStruct((N_PAD, D), jnp.float32)),
    )(p, norm, w, b, i_node)


def _gcn_i_body(p_ref, norm_ref, w_ref, b_ref, i_out):
    agg = (p_ref[0] + p_ref[1]) * norm_ref[...]
    r = jnp.dot(agg, w_ref[...], preferred_element_type=jnp.float32) + b_ref[...]
    i_out[...] = jnp.maximum(r, 0.0)


def _gcn_i_call(p, norm, w, b):
    return pl.pallas_call(
        _gcn_i_body,
        grid=(N_PAD // RB,),
        in_specs=[
            pl.BlockSpec((NC, RB, D), lambda i: (0, i, 0)),
            pl.BlockSpec((RB, 1), lambda i: (i, 0)),
            pl.BlockSpec((D, D), lambda i: (0, 0)),
            pl.BlockSpec((1, D), lambda i: (0, 0)),
        ],
        out_specs=pl.BlockSpec((RB, D), lambda i: (i, 0)),
        out_shape=jax.ShapeDtypeStruct((N_PAD, D), jnp.float32),
    )(p, norm, w, b)


def _leaky(x):
    return jnp.where(x >= 0, x, 0.01 * x)


def _mlp_body(p_ref, w1, b1, w2, b2, w3, b3, out_ref):
    d = p_ref[0, :N_D, :] + p_ref[1, :N_D, :]
    h = _leaky(jnp.dot(d, w1[...], preferred_element_type=jnp.float32) + b1[...])
    h = _leaky(jnp.dot(h, w2[...], preferred_element_type=jnp.float32) + b2[...])
    out_ref[...] = jnp.dot(h, w3[...], preferred_element_type=jnp.float32) + b3[...]


def _mlp_call(d_p, w1, b1, w2, b2, w3, b3):
    return pl.pallas_call(
        _mlp_body,
        out_shape=jax.ShapeDtypeStruct((N_D, 1), jnp.float32),
    )(d_p, w1, b1, w2, b2, w3, b3)


def kernel(r_node, i_node, r2r_edge, d2d_edge, i2i_src, i2i_dst, i2d_src, i2d_dst,
           W_r2r, b_r2r, W_i2i, b_i2i, W_s1, b_s1, W_s2, b_s2, W_s3, b_s3):
    r_node = jnp.pad(r_node, ((0, N_PAD - N_I), (0, 0)))
    i_node = jnp.pad(i_node, ((0, N_PAD - N_I), (0, 0)))
    src_r = i2i_src.reshape(NW, NBLK, IB, C)
    dst_r = i2i_dst.reshape(NW, NBLK, IB, C)
    pad = ID_PAD - E_ID
    id_src = jnp.concatenate([i2d_src, jnp.zeros((pad,), jnp.int32)]
                             ).reshape(NW, ID_CHUNKS, C)
    id_dst = jnp.concatenate([i2d_dst, jnp.full((pad,), N_D, jnp.int32)]
                             ).reshape(NW, ID_CHUNKS, C)

    deg_p = _deg_kernel(dst_r)
    norm, h = _prep_call(deg_p, r_node)

    for l in range(L):
        p = _agg_kernel(h, src_r, dst_r)
        h, hi = _gcn_r_call(p, norm, W_r2r[l], b_r2r[l].reshape(1, D), i_node)
        p = _agg_kernel(hi, src_r, dst_r)
        i_node = _gcn_i_call(p, norm, W_i2i[l], b_i2i[l].reshape(1, D))

    d_p = _i2d_kernel(i_node, id_src, id_dst)
    return _mlp_call(d_p, W_s1, b_s1.reshape(1, -1), W_s2, b_s2.reshape(1, -1),
                     W_s3, b_s3.reshape(1, -1))
